# baseline (speedup 1.0000x reference)
import functools

import jax
import jax.numpy as jnp
from jax import lax
from jax.experimental import pallas as pl
from jax.experimental.pallas import tpu as pltpu
from jax.experimental.pallas import tpu_sc as plsc

_NW = 32
_NB = 4


def _make_lookup(B, S, D):
    BW = B // _NW
    n_grp = S // _NB
    mesh = plsc.VectorSubcoreMesh(core_axis_name="c", subcore_axis_name="s")

    @functools.partial(
        pl.kernel,
        out_type=jax.ShapeDtypeStruct((S, D, B), jnp.float32),
        mesh=mesh,
        scratch_types=[
            pltpu.VMEM((S, BW), jnp.int32),
            pltpu.VMEM((_NB, BW, D), jnp.float32),
            pltpu.VMEM((_NB, D, BW), jnp.float32),
            pltpu.SemaphoreType.DMA((_NB,)),
            pltpu.SemaphoreType.DMA((_NB,)),
        ],
        compiler_params=pltpu.CompilerParams(
            use_tc_tiling_on_sc=False, needs_layout_passes=False
        ),
    )
    def lookup(idsT_hbm, tbl_hbm, out_hbm, idsv, gbuf, tbuf, gsem, ssem):
        wid = lax.axis_index("s") * 2 + lax.axis_index("c")
        b0 = wid * BW

        pltpu.sync_copy(idsT_hbm.at[:, pl.ds(b0, BW)], idsv)

        def gather_desc(s, b):
            return pltpu.make_async_copy(
                tbl_hbm.at[idsv.at[s]], gbuf.at[b], gsem.at[b]
            )

        def scatter_desc(s, b):
            return pltpu.make_async_copy(
                tbuf.at[b], out_hbm.at[s, :, pl.ds(b0, BW)], ssem.at[b]
            )

        lanes = lax.iota(jnp.int32, 16)
        diag = [(lanes + d) & 15 for d in range(16)]
        col = [lanes + 16 * c for c in range(D // 16)]

        def transpose(b):
            def ablock(a, carry):
                a16 = a * 16
                for c in range(D // 16):
                    for d in range(16):
                        row = diag[d] + a16
                        v = plsc.load_gather(gbuf.at[b], [row, col[c]])
                        plsc.store_scatter(tbuf.at[b], [col[c], row], v)
                return carry

            lax.fori_loop(0, BW // 16, ablock, 0)

        for b in range(_NB):
            gather_desc(b, b).start()

        def body(g, carry):
            s0 = g * _NB
            for b in range(_NB):
                s = s0 + b
                gather_desc(s, b).wait()

                @pl.when(g > 0)
                def _():
                    scatter_desc(s - _NB, b).wait()

                transpose(b)

                @pl.when(g < n_grp - 1)
                def _():
                    gather_desc(s + _NB, b).start()

                scatter_desc(s, b).start()
            return carry

        lax.fori_loop(0, n_grp, body, 0)

        for b in range(_NB):
            scatter_desc(S - _NB + b, b).wait()

    return lookup


def kernel(input_ids, word_embeddings):
    B, S = input_ids.shape
    V, D = word_embeddings.shape
    idsT = input_ids.T.astype(jnp.int32)
    out_phys = _make_lookup(B, S, D)(idsT, word_embeddings)
    return jnp.transpose(out_phys, (2, 0, 1))


# speedup vs baseline: 1.0179x; 1.0179x over previous
import functools

import jax
import jax.numpy as jnp
from jax import lax
from jax.experimental import pallas as pl
from jax.experimental.pallas import tpu as pltpu
from jax.experimental.pallas import tpu_sc as plsc

_NW = 32
_NB = 2


def _make_lookup(B, S, D):
    BW = B // _NW
    n_grp = S // _NB
    mesh = plsc.VectorSubcoreMesh(core_axis_name="c", subcore_axis_name="s")

    @functools.partial(
        pl.kernel,
        out_type=jax.ShapeDtypeStruct((S, D, B), jnp.float32),
        mesh=mesh,
        scratch_types=[
            pltpu.VMEM((S, BW), jnp.int32),
            pltpu.VMEM((_NB, BW, D), jnp.float32),
            pltpu.VMEM((_NB, D, BW), jnp.float32),
            pltpu.SemaphoreType.DMA((_NB,)),
            pltpu.SemaphoreType.DMA((_NB,)),
        ],
        compiler_params=pltpu.CompilerParams(
            use_tc_tiling_on_sc=False, needs_layout_passes=False
        ),
    )
    def lookup(idsT_hbm, tbl_hbm, out_hbm, idsv, gbuf, tbuf, gsem, ssem):
        wid = lax.axis_index("s") * 2 + lax.axis_index("c")
        b0 = wid * BW

        pltpu.sync_copy(idsT_hbm.at[:, pl.ds(b0, BW)], idsv)

        def gather_desc(s, b):
            return pltpu.make_async_copy(
                tbl_hbm.at[idsv.at[s]], gbuf.at[b], gsem.at[b]
            )

        def scatter_desc(s, b):
            return pltpu.make_async_copy(
                tbuf.at[b], out_hbm.at[s, :, pl.ds(b0, BW)], ssem.at[b]
            )

        lanes = lax.iota(jnp.int32, 16)
        diag = [(lanes + d) & 15 for d in range(16)]
        col = [lanes + 16 * c for c in range(D // 16)]

        def transpose(b):
            def ablock(a, carry):
                a16 = a * 16
                for c in range(D // 16):
                    for d in range(16):
                        row = diag[d] + a16
                        v = plsc.load_gather(gbuf.at[b], [row, col[c]])
                        plsc.store_scatter(tbuf.at[b], [col[c], row], v)
                return carry

            lax.fori_loop(0, BW // 16, ablock, 0)

        for b in range(_NB):
            gather_desc(b, b).start()

        def body(g, carry):
            s0 = g * _NB
            for b in range(_NB):
                s = s0 + b
                gather_desc(s, b).wait()

                @pl.when(g > 0)
                def _():
                    scatter_desc(s - _NB, b).wait()

                transpose(b)

                @pl.when(g < n_grp - 1)
                def _():
                    gather_desc(s + _NB, b).start()

                scatter_desc(s, b).start()
            return carry

        lax.fori_loop(0, n_grp, body, 0)

        for b in range(_NB):
            scatter_desc(S - _NB + b, b).wait()

    return lookup


def kernel(input_ids, word_embeddings):
    B, S = input_ids.shape
    V, D = word_embeddings.shape
    idsT = input_ids.T.astype(jnp.int32)
    out_phys = _make_lookup(B, S, D)(idsT, word_embeddings)
    return jnp.transpose(out_phys, (2, 0, 1))
